# Initial kernel scaffold; baseline (speedup 1.0000x reference)
import jax
import jax.numpy as jnp
from jax.experimental import pallas as pl


def kernel(x, time_day, time_week):
    raise NotImplementedError("write your pallas kernel here")


# speedup vs baseline: 2.7600x; 2.7600x over previous
import functools

import jax
import jax.numpy as jnp
from jax import lax
from jax.experimental import pallas as pl
from jax.experimental.pallas import tpu as pltpu
from jax.experimental.pallas import tpu_sc as plsc

_TIME = 288
_WPAD = 8
_CTROW = _TIME * _WPAD
_B, _T, _N, _F = 64, 12, 2048, 128
_NC, _NS, _L = 2, 16, 16
_NW = _NC * _NS
_FPW = _F // _NW
_BPS = _B // _NS
_CHUNKS = _N // _L


def _body(xs_hbm, td_hbm, tw_hbm, out_hbm,
          td_v, tw_v, ct_v, xbuf_v, idx_v, ib_v, orow_v, shared_idx):
    cid = lax.axis_index("c")
    sid = lax.axis_index("s")
    wid = sid * _NC + cid
    f0 = wid * _FPW
    iota = lax.iota(jnp.int32, _L)

    pltpu.sync_copy(td_hbm, td_v)
    pltpu.sync_copy(tw_hbm, tw_v)

    def _build(c, carry):
        ivec = c * _L + iota
        dvec = lax.shift_right_logical(ivec, 3)
        wvec = jnp.minimum(ivec & (_WPAD - 1), 6)
        for fl in range(_FPW):
            tdcol = plsc.load_gather(td_v, [dvec * _F + (f0 + fl)])
            twcol = plsc.load_gather(tw_v, [wvec * _F + (f0 + fl)])
            ct_v[pl.ds(fl * _CTROW + c * _L, _L)] = tdcol + twcol
        return carry

    lax.fori_loop(0, _CTROW // _L, _build, 0)

    for i in range(_BPS):
        bb = sid * _BPS + i
        pltpu.sync_copy(xs_hbm.at[bb], xbuf_v)

        def _decode(c, carry):
            nvec3 = (c * _L + iota) * 3
            frac = plsc.load_gather(xbuf_v, [nvec3 + 1])
            wraw = plsc.load_gather(xbuf_v, [nvec3 + 2])
            d = (frac * float(_TIME)).astype(jnp.int32)
            d = jnp.minimum(jnp.maximum(d, 0), _TIME - 1)
            w = wraw.astype(jnp.int32)
            w = jnp.minimum(jnp.maximum(w, 0), 6)
            idx_v[pl.ds(c * _L, _L)] = d * _WPAD + w
            return carry

        lax.fori_loop(0, _CHUNKS, _decode, 0)
        pltpu.sync_copy(idx_v, shared_idx.at[bb])

    plsc.subcore_barrier()

    def _batch(b, carry):
        pltpu.sync_copy(shared_idx.at[b], ib_v)

        def _chunk(c, carry2):
            base = c * _L
            cidx = ib_v[pl.ds(base, _L)]
            for fl in range(_FPW):
                v = plsc.load_gather(ct_v, [cidx + fl * _CTROW])
                orow_v[fl, pl.ds(base, _L)] = v
            return carry2

        lax.fori_loop(0, _CHUNKS, _chunk, 0)
        pltpu.sync_copy(orow_v, out_hbm.at[b, pl.ds(f0, _FPW)])
        return carry

    lax.fori_loop(0, _B, _batch, 0)


@jax.jit
def _run(xs, time_day_flat, time_week_flat):
    mesh = plsc.VectorSubcoreMesh(core_axis_name="c", subcore_axis_name="s")
    k = functools.partial(
        pl.kernel,
        out_type=jax.ShapeDtypeStruct((_B, _F, _N), jnp.float32),
        mesh=mesh,
        compiler_params=pltpu.CompilerParams(needs_layout_passes=False),
        scratch_types=[
            pltpu.VMEM((_TIME * _F,), jnp.float32),
            pltpu.VMEM((7 * _F,), jnp.float32),
            pltpu.VMEM((_FPW * _CTROW,), jnp.float32),
            pltpu.VMEM((_N * 3,), jnp.float32),
            pltpu.VMEM((_N,), jnp.int32),
            pltpu.VMEM((_N,), jnp.int32),
            pltpu.VMEM((_FPW, _N), jnp.float32),
            pltpu.VMEM_SHARED((_B, _N), jnp.int32),
        ],
    )(_body)
    return k(xs, time_day_flat, time_week_flat)


def kernel(x, time_day, time_week):
    xs = x[:, -1].reshape(_B, _N * 3)
    out = _run(xs, time_day.reshape(-1), time_week.reshape(-1))
    return out[..., None]
